# baseline (speedup 1.0000x reference)
import functools

import jax
import jax.numpy as jnp
from jax import lax
from jax.experimental import pallas as pl
from jax.experimental.pallas import tpu as pltpu
from jax.experimental.pallas import tpu_sc as plsc

N_NODES = 10000
N_EDGES = 320000
D_FEAT = 128
D_OUT = 3 + D_FEAT
FPAD = 136

NC = 2
NS = 16
NW = NC * NS

BLK = 128
CHUNK = BLK
NCHUNK = N_EDGES // CHUNK
TROWS = (D_OUT + 7) // 8
PW = 8
NPAIR = (NCHUNK // NW + 2) // 2


def _body(feat_hbm, posd_hbm, src_hbm, dst_hbm, out_hbm,
          src_v0, dst_v0, f_v0, ps_v0, pd_v0, t_v0, gsem0, wsem0,
          src_v1, dst_v1, f_v1, ps_v1, pd_v1, t_v1, gsem1, wsem1):
    wid = lax.axis_index("s") * NC + lax.axis_index("c")
    base_rows = lax.iota(jnp.int32, 16)
    bufs = [
        (src_v0, dst_v0, f_v0, ps_v0, pd_v0, t_v0, gsem0, wsem0),
        (src_v1, dst_v1, f_v1, ps_v1, pd_v1, t_v1, gsem1, wsem1),
    ]

    def start(b, j):
        src_v, dst_v, f_v, ps_v, pd_v, t_v, gsem, wsem = bufs[b]

        @pl.when(j < NCHUNK)
        def _():
            pltpu.sync_copy(src_hbm.at[pl.ds(j, 1)], src_v)
            pltpu.sync_copy(dst_hbm.at[pl.ds(j, 1)], dst_v)
            pltpu.async_copy(feat_hbm.at[src_v.at[0]], f_v, gsem)
            pltpu.async_copy(posd_hbm.at[src_v.at[0]], ps_v, gsem)
            pltpu.async_copy(posd_hbm.at[dst_v.at[0]], pd_v, gsem)

    def finish(b, j, drain_write):
        src_v, dst_v, f_v, ps_v, pd_v, t_v, gsem, wsem = bufs[b]

        @pl.when(jnp.logical_and(j < NCHUNK, drain_write))
        def _():
            pltpu.make_async_copy(
                t_v, out_hbm.at[:, pl.ds(0, 1)], wsem).wait()

        @pl.when(j < NCHUNK)
        def _():
            pltpu.make_async_copy(
                feat_hbm.at[pl.ds(0, CHUNK)], f_v, gsem).wait()
            pltpu.make_async_copy(
                posd_hbm.at[pl.ds(0, CHUNK)], ps_v, gsem).wait()
            pltpu.make_async_copy(
                posd_hbm.at[pl.ds(0, CHUNK)], pd_v, gsem).wait()

            zeros = jnp.zeros((16,), jnp.int32)

            for c in range(3):
                cols = jnp.full((16,), c, jnp.int32)
                for g in range(CHUNK // 16):
                    rows = base_rows + (g * 16)
                    a = plsc.load_gather(ps_v, [rows, cols])
                    bb = plsc.load_gather(pd_v, [rows, cols])
                    t_v[0, 0, c, pl.ds(g * 16, 16)] = a - bb

            def win_body(w, carry):
                c0 = w * 16
                for eg in range(CHUNK // 16):
                    rows = base_rows + (eg * 16)
                    for r0 in range(16):
                        m = (base_rows + r0) & 15
                        cols = c0 + m
                        c_out = cols + 3
                        cdiv = c_out >> 3
                        cmod = c_out & 7
                        v = plsc.load_gather(f_v, [rows, cols])
                        plsc.store_scatter(t_v, [cdiv, zeros, cmod, rows], v)
                return carry

            lax.fori_loop(0, D_FEAT // 16, win_body, 0)

            pltpu.async_copy(t_v, out_hbm.at[:, pl.ds(j, 1)], wsem)

    start(0, wid)
    start(1, wid + NW)

    def pair_body(tt, carry):
        jA = wid + (2 * tt) * NW
        jB = jA + NW
        finish(0, jA, tt >= 1)
        start(0, jA + 2 * NW)
        finish(1, jB, tt >= 1)
        start(1, jB + 2 * NW)
        return carry

    lax.fori_loop(0, NPAIR, pair_body, 0)

    for b in range(2):
        t_v, wsem = bufs[b][5], bufs[b][7]
        pltpu.make_async_copy(t_v, out_hbm.at[:, pl.ds(0, 1)], wsem).wait()


@jax.jit
def _run(feat136, posd, src2d, dst2d):
    mesh = plsc.VectorSubcoreMesh(
        core_axis_name="c", subcore_axis_name="s",
        num_cores=NC, num_subcores=NS)
    bufset = [
        pltpu.VMEM((1, CHUNK), jnp.int32),
        pltpu.VMEM((1, CHUNK), jnp.int32),
        pltpu.VMEM((CHUNK, FPAD), jnp.float32),
        pltpu.VMEM((CHUNK, PW), jnp.float32),
        pltpu.VMEM((CHUNK, PW), jnp.float32),
        pltpu.VMEM((TROWS, 1, 8, BLK), jnp.float32),
        pltpu.SemaphoreType.DMA,
        pltpu.SemaphoreType.DMA,
    ]
    f = pl.kernel(
        _body,
        out_type=jax.ShapeDtypeStruct((TROWS, NCHUNK, 8, BLK), jnp.float32),
        mesh=mesh,
        scratch_types=bufset + bufset,
        compiler_params=pltpu.CompilerParams(
            use_tc_tiling_on_sc=False, needs_layout_passes=False,
            disable_bounds_checks=True),
    )
    return f(feat136, posd, src2d, dst2d)


def kernel(pos, feat, edge_index):
    feat136 = jnp.pad(feat, ((0, 0), (0, FPAD - D_FEAT)))
    posd = jnp.pad(pos, ((0, 0), (0, PW - 3)))
    src2d = edge_index[0].astype(jnp.int32).reshape(NCHUNK, CHUNK)
    dst2d = edge_index[1].astype(jnp.int32).reshape(NCHUNK, CHUNK)
    out4 = _run(feat136, posd, src2d, dst2d)
    out136 = out4.transpose(1, 3, 0, 2).reshape(N_EDGES, TROWS * 8)
    return out136[:, :D_OUT]


# speedup vs baseline: 1.1110x; 1.1110x over previous
import jax
import jax.numpy as jnp
from jax import lax
from jax.experimental import pallas as pl
from jax.experimental.pallas import tpu as pltpu
from jax.experimental.pallas import tpu_sc as plsc

N_NODES = 10000
N_EDGES = 320000
D_FEAT = 128
D_OUT = 3 + D_FEAT
FPAD = 136

NC = 2
NS = 16
NW = NC * NS

BLK = 128
CHUNK = BLK
NCHUNK = N_EDGES // CHUNK
TROWS = (D_OUT + 7) // 8
PW = 8
NPAIR = (NCHUNK // NW + 2) // 2


def _body(feat_hbm, posd_hbm, src_hbm, dst_hbm, out_hbm,
          src_v0, dst_v0, f_v0, ps_v0, pd_v0, t_v0, gsem0, wsem0,
          src_v1, dst_v1, f_v1, ps_v1, pd_v1, t_v1, gsem1, wsem1):
    wid = lax.axis_index("s") * NC + lax.axis_index("c")
    base_rows = lax.iota(jnp.int32, 16)
    bufs = [
        (src_v0, dst_v0, f_v0, ps_v0, pd_v0, t_v0, gsem0, wsem0),
        (src_v1, dst_v1, f_v1, ps_v1, pd_v1, t_v1, gsem1, wsem1),
    ]

    def start(b, j):
        src_v, dst_v, f_v, ps_v, pd_v, t_v, gsem, wsem = bufs[b]

        @pl.when(j < NCHUNK)
        def _():
            pltpu.sync_copy(src_hbm.at[pl.ds(j, 1)], src_v)
            pltpu.sync_copy(dst_hbm.at[pl.ds(j, 1)], dst_v)
            pltpu.async_copy(feat_hbm.at[src_v.at[0]], f_v, gsem)
            pltpu.async_copy(posd_hbm.at[src_v.at[0]], ps_v, gsem)
            pltpu.async_copy(posd_hbm.at[dst_v.at[0]], pd_v, gsem)

    def finish(b, j, drain_write):
        src_v, dst_v, f_v, ps_v, pd_v, t_v, gsem, wsem = bufs[b]

        @pl.when(jnp.logical_and(j < NCHUNK, drain_write))
        def _():
            pltpu.make_async_copy(
                t_v, out_hbm.at[:, pl.ds(0, 1)], wsem).wait()

        @pl.when(j < NCHUNK)
        def _():
            pltpu.make_async_copy(
                feat_hbm.at[pl.ds(0, CHUNK)], f_v, gsem).wait()
            pltpu.make_async_copy(
                posd_hbm.at[pl.ds(0, CHUNK)], ps_v, gsem).wait()
            pltpu.make_async_copy(
                posd_hbm.at[pl.ds(0, CHUNK)], pd_v, gsem).wait()

            zeros = jnp.zeros((16,), jnp.int32)

            for c in range(3):
                cols = jnp.full((16,), c, jnp.int32)
                for g in range(CHUNK // 16):
                    rows = base_rows + (g * 16)
                    a = plsc.load_gather(ps_v, [rows, cols])
                    bb = plsc.load_gather(pd_v, [rows, cols])
                    t_v[0, 0, c, pl.ds(g * 16, 16)] = a - bb

            def win_body(w, carry):
                c0 = w * 16
                for r0 in range(16):
                    m = (base_rows + r0) & 15
                    cols = c0 + m
                    c_out = cols + 3
                    cdiv = c_out >> 3
                    cmod = c_out & 7
                    for eg in range(CHUNK // 16):
                        rows = base_rows + (eg * 16)
                        v = plsc.load_gather(f_v, [rows, cols])
                        plsc.store_scatter(t_v, [cdiv, zeros, cmod, rows], v)
                return carry

            lax.fori_loop(0, D_FEAT // 16, win_body, 0)

            pltpu.async_copy(t_v, out_hbm.at[:, pl.ds(j, 1)], wsem)

    start(0, wid)
    start(1, wid + NW)

    def pair_body(tt, carry):
        jA = wid + (2 * tt) * NW
        jB = jA + NW
        finish(0, jA, tt >= 1)
        start(0, jA + 2 * NW)
        finish(1, jB, tt >= 1)
        start(1, jB + 2 * NW)
        return carry

    lax.fori_loop(0, NPAIR, pair_body, 0)

    for b in range(2):
        t_v, wsem = bufs[b][5], bufs[b][7]
        pltpu.make_async_copy(t_v, out_hbm.at[:, pl.ds(0, 1)], wsem).wait()


@jax.jit
def _run(feat136, posd, src2d, dst2d):
    mesh = plsc.VectorSubcoreMesh(
        core_axis_name="c", subcore_axis_name="s",
        num_cores=NC, num_subcores=NS)
    bufset = [
        pltpu.VMEM((1, CHUNK), jnp.int32),
        pltpu.VMEM((1, CHUNK), jnp.int32),
        pltpu.VMEM((CHUNK, FPAD), jnp.float32),
        pltpu.VMEM((CHUNK, PW), jnp.float32),
        pltpu.VMEM((CHUNK, PW), jnp.float32),
        pltpu.VMEM((TROWS, 1, 8, BLK), jnp.float32),
        pltpu.SemaphoreType.DMA,
        pltpu.SemaphoreType.DMA,
    ]
    f = pl.kernel(
        _body,
        out_type=jax.ShapeDtypeStruct((TROWS, NCHUNK, 8, BLK), jnp.float32),
        mesh=mesh,
        scratch_types=bufset + bufset,
        compiler_params=pltpu.CompilerParams(
            use_tc_tiling_on_sc=False, needs_layout_passes=False,
            disable_bounds_checks=True),
    )
    return f(feat136, posd, src2d, dst2d)


def kernel(pos, feat, edge_index):
    feat136 = jnp.pad(feat, ((0, 0), (0, FPAD - D_FEAT)))
    posd = jnp.pad(pos, ((0, 0), (0, PW - 3)))
    src2d = edge_index[0].astype(jnp.int32).reshape(NCHUNK, CHUNK)
    dst2d = edge_index[1].astype(jnp.int32).reshape(NCHUNK, CHUNK)
    out4 = _run(feat136, posd, src2d, dst2d)
    out136 = out4.transpose(1, 3, 0, 2).reshape(N_EDGES, TROWS * 8)
    return out136[:, :D_OUT]
